# Initial kernel scaffold; baseline (speedup 1.0000x reference)
import jax
import jax.numpy as jnp
from jax.experimental import pallas as pl


def kernel(node_features, edge_features, edge_indexes, W1, b1, W2, b2, W3, b3, Wv, bv, We, be):
    raise NotImplementedError("write your pallas kernel here")


# speedup vs baseline: 8.4893x; 8.4893x over previous
import functools

import jax
import jax.numpy as jnp
from jax import lax
from jax.experimental import pallas as pl
from jax.experimental.pallas import tpu as pltpu
from jax.experimental.pallas import tpu_sc as plsc

_NC = 2
_NS = 16
_LANES = 16


def _k1a_body(nf_ref, p_ref, q_ref, o_ref):
    o_ref[...] = (
        jnp.dot(nf_ref[...], p_ref[...], preferred_element_type=jnp.float32)
        + q_ref[...]
    )


def _k1b_body(ef_ref, mw_ref, o_ref):
    o_ref[...] = jnp.dot(ef_ref[...], mw_ref[...],
                         preferred_element_type=jnp.float32)


def _k2_body(a10_ref, a11_ref, a20_ref, a21_ref, nf_ref,
             A_ref, B_ref, Wc_ref, c0_ref, o_ref):
    accV = a10_ref[...] + a11_ref[...]
    s2 = a20_ref[...] + a21_ref[...]
    accE = s2[:, :16]
    dV = jnp.maximum(s2[:, 16:17], 1e-30)
    dE = jnp.maximum(s2[:, 17:18], 1e-30)
    deg = s2[:, 18:19]
    emb = (
        jnp.dot(accV / dV, A_ref[...], preferred_element_type=jnp.float32)
        + jnp.dot(accE / dE, B_ref[...], preferred_element_type=jnp.float32)
        + deg * (jnp.dot(nf_ref[...], Wc_ref[...],
                         preferred_element_type=jnp.float32) + c0_ref[...])
    )
    m = jnp.max(emb, axis=1, keepdims=True)
    ex = jnp.exp(emb - m)
    o_ref[...] = ex / jnp.sum(ex, axis=1, keepdims=True)


def _make_sc_scalar_kernel(N, E, DN, DE, NPAD):
    NT = _NC * _NS
    EP = E // NT
    C = 16
    NCHUNK = EP // C
    NP4 = NPAD // 4
    SR2 = NP4 // _NS

    mesh = plsc.VectorSubcoreMesh(core_axis_name="c", subcore_axis_name="s")

    def body(ef, u, v, s0, s1, s2, s3, ee, zst, evw, acc2,
             u_v, v_v, ee_v, s0_v, s1_v, s2_v, s3_v,
             ev_slab, pay2, ef_v, acc2_sh):
        cid = lax.axis_index("c")
        sid = lax.axis_index("s")
        base = (cid * _NS + sid) * EP

        pltpu.sync_copy(zst.at[pl.ds(0, SR2)], acc2_sh.at[pl.ds(sid * SR2, SR2)])

        pltpu.sync_copy(u.at[pl.ds(base, EP)], u_v)
        pltpu.sync_copy(v.at[pl.ds(base, EP)], v_v)
        pltpu.sync_copy(ee.at[pl.ds(base, EP)], ee_v)
        pltpu.sync_copy(s0, s0_v)
        pltpu.sync_copy(s1, s1_v)
        pltpu.sync_copy(s2, s2_v)
        pltpu.sync_copy(s3, s3_v)

        zv = jnp.zeros((_LANES,), jnp.float32)
        for j in range(C):
            for r in range(8):
                pay2[j, pl.ds(r * _LANES, _LANES)] = zv

        plsc.subcore_barrier()

        row_iota = lax.iota(jnp.int32, _LANES)
        ones16 = jnp.ones((_LANES,), jnp.float32)

        def chunk(k, carry):
            off = k * C
            u16 = u_v[pl.ds(off, _LANES)]
            v16 = v_v[pl.ds(off, _LANES)]
            sVu = plsc.load_gather(s0_v, [u16])
            sVv = plsc.load_gather(s1_v, [v16])
            sEu = plsc.load_gather(s2_v, [u16])
            sEv = plsc.load_gather(s3_v, [v16])
            ee16 = ee_v[pl.ds(off, _LANES)]
            aV = sVu + sVv
            aV = jnp.where(aV >= 0.0, aV, aV * 0.01)
            aE = sEu + sEv + ee16
            aE = jnp.where(aE >= 0.0, aE, aE * 0.01)
            eV = jnp.exp(aV)
            eEx = jnp.exp(aE)
            ev_slab[pl.ds(off, _LANES)] = eV

            pltpu.sync_copy(ef.at[pl.ds(base + off, C)], ef_v)

            gcol = lax.shift_left(jnp.bitwise_and(u16, 3), 5)
            for f in range(DE):
                colf = jnp.full((_LANES,), f, jnp.int32)
                efc = plsc.load_gather(ef_v, [row_iota, colf])
                plsc.store_scatter(pay2, [row_iota, gcol + f], efc * eEx)
            plsc.store_scatter(pay2, [row_iota, gcol + DE], eV)
            plsc.store_scatter(pay2, [row_iota, gcol + (DE + 1)], eEx)
            plsc.store_scatter(pay2, [row_iota, gcol + (DE + 2)], ones16)

            u4 = lax.shift_right_logical(u16, 2)
            pltpu.sync_copy(pay2, acc2_sh.at[u4], add=True)

            for f in range(DE):
                plsc.store_scatter(pay2, [row_iota, gcol + f], zv)
            plsc.store_scatter(pay2, [row_iota, gcol + DE], zv)
            plsc.store_scatter(pay2, [row_iota, gcol + (DE + 1)], zv)
            plsc.store_scatter(pay2, [row_iota, gcol + (DE + 2)], zv)
            return carry

        lax.fori_loop(0, NCHUNK, chunk, 0)

        pltpu.sync_copy(ev_slab, evw.at[pl.ds(base, EP)])
        plsc.subcore_barrier()
        pltpu.sync_copy(acc2_sh.at[pl.ds(sid * SR2, SR2)],
                        acc2.at[cid, pl.ds(sid * SR2, SR2)])

    return functools.partial(
        pl.kernel,
        out_type=(
            jax.ShapeDtypeStruct((E,), jnp.float32),
            jax.ShapeDtypeStruct((_NC, NP4, 128), jnp.float32),
        ),
        mesh=mesh,
        compiler_params=pltpu.CompilerParams(needs_layout_passes=False),
        scratch_types=[
            pltpu.VMEM((EP,), jnp.int32),
            pltpu.VMEM((EP,), jnp.int32),
            pltpu.VMEM((EP,), jnp.float32),
            pltpu.VMEM((N,), jnp.float32),
            pltpu.VMEM((N,), jnp.float32),
            pltpu.VMEM((N,), jnp.float32),
            pltpu.VMEM((N,), jnp.float32),
            pltpu.VMEM((EP,), jnp.float32),
            pltpu.VMEM((C, 128), jnp.float32),
            pltpu.VMEM((C, DE), jnp.float32),
            pltpu.VMEM_SHARED((NP4, 128), jnp.float32),
        ],
    )(body)


def _make_sc_rows_kernel(N, E, DN, NPAD):
    NT = _NC * _NS
    EP = E // NT
    C = 16
    NCHUNK = EP // C
    SR1 = NPAD // _NS

    mesh = plsc.VectorSubcoreMesh(core_axis_name="c", subcore_axis_name="s")

    def body(nf, u, v, evw, zst, acc1,
             u_v, v_v, ev_slab, pay1, ev_v, acc1_sh):
        cid = lax.axis_index("c")
        sid = lax.axis_index("s")
        base = (cid * _NS + sid) * EP

        pltpu.sync_copy(zst, acc1_sh.at[pl.ds(sid * SR1, SR1)])
        pltpu.sync_copy(u.at[pl.ds(base, EP)], u_v)
        pltpu.sync_copy(v.at[pl.ds(base, EP)], v_v)
        pltpu.sync_copy(evw.at[pl.ds(base, EP)], ev_slab)
        plsc.subcore_barrier()

        def chunk(k, carry):
            off = k * C
            u16 = u_v[pl.ds(off, _LANES)]
            v16 = v_v[pl.ds(off, _LANES)]
            eV = ev_slab[pl.ds(off, _LANES)]

            pltpu.sync_copy(nf.at[v16], pay1)
            ev_v[...] = eV
            for j in range(C):
                sv = plsc.load_gather(ev_v, [jnp.full((_LANES,), j, jnp.int32)])
                for r in range(DN // _LANES):
                    pay1[j, pl.ds(r * _LANES, _LANES)] = (
                        pay1[j, pl.ds(r * _LANES, _LANES)] * sv)

            pltpu.sync_copy(pay1, acc1_sh.at[u16], add=True)
            return carry

        lax.fori_loop(0, NCHUNK, chunk, 0)

        plsc.subcore_barrier()
        pltpu.sync_copy(acc1_sh.at[pl.ds(sid * SR1, SR1)],
                        acc1.at[cid, pl.ds(sid * SR1, SR1)])

    return functools.partial(
        pl.kernel,
        out_type=jax.ShapeDtypeStruct((_NC, NPAD, 128), jnp.float32),
        mesh=mesh,
        compiler_params=pltpu.CompilerParams(needs_layout_passes=False),
        scratch_types=[
            pltpu.VMEM((EP,), jnp.int32),
            pltpu.VMEM((EP,), jnp.int32),
            pltpu.VMEM((EP,), jnp.float32),
            pltpu.VMEM((C, DN), jnp.float32),
            pltpu.VMEM((_LANES,), jnp.float32),
            pltpu.VMEM_SHARED((NPAD, 128), jnp.float32),
        ],
    )(body)


def kernel(node_features, edge_features, edge_indexes,
           W1, b1, W2, b2, W3, b3, Wv, bv, We, be):
    N, DN = node_features.shape
    E, DE = edge_features.shape
    DO = W3.shape[1]
    u = edge_indexes[0]
    v = edge_indexes[1]
    NPAD = -(-N // 256) * 256

    C4 = jnp.stack([Wv[:DN, 0], Wv[DN:, 0], We[:DN, 0], We[DN:2 * DN, 0]],
                   axis=1)
    P = W1 @ C4
    q = b1 @ C4
    q = q.at[0].add(bv[0])
    q = q.at[2].add(b2 @ We[2 * DN:, 0] + be[0])
    P_pad = jnp.zeros((DN, 128), jnp.float32).at[:, :4].set(P)
    q_pad = jnp.zeros((1, 128), jnp.float32).at[0, :4].set(q)
    w2e = W2 @ We[2 * DN:, 0]
    A = W1 @ W3[:DN]
    B = W2 @ W3[DN:DN + DE]
    W3c = W3[DN + DE:]
    c0 = (b1 @ W3[:DN] + b2 @ W3[DN:DN + DE] + b3)[None, :]

    S_full = pl.pallas_call(
        _k1a_body,
        out_shape=jax.ShapeDtypeStruct((N, 128), jnp.float32),
    )(node_features, P_pad, q_pad)
    s0, s1, s2, s3 = (S_full[:, 0], S_full[:, 1], S_full[:, 2], S_full[:, 3])

    gsel = jnp.arange(128) // DE
    Mw = jnp.where(gsel[:, None] == jnp.arange(8)[None, :],
                   jnp.tile(w2e, 8)[:, None], 0.0)
    ef2 = edge_features.reshape(E // 8, 128)
    eE = pl.pallas_call(
        _k1b_body,
        out_shape=jax.ShapeDtypeStruct((E // 8, 8), jnp.float32),
    )(ef2, Mw).reshape(E)

    zstripe = jnp.zeros((NPAD // _NS, 128), jnp.float32)
    evw, acc2 = _make_sc_scalar_kernel(N, E, DN, DE, NPAD)(
        edge_features, u, v, s0, s1, s2, s3, eE, zstripe)
    acc1 = _make_sc_rows_kernel(N, E, DN, NPAD)(
        node_features, u, v, evw, zstripe)
    acc2r = acc2.reshape(_NC, NPAD, 32)

    nf_pad = jnp.pad(node_features, ((0, NPAD - N), (0, 0)))
    out = pl.pallas_call(
        _k2_body,
        out_shape=jax.ShapeDtypeStruct((NPAD, DO), jnp.float32),
    )(acc1[0], acc1[1], acc2r[0], acc2r[1], nf_pad, A, B, W3c, c0)
    return out[:N]


# speedup vs baseline: 15.0851x; 1.7769x over previous
import functools

import jax
import jax.numpy as jnp
from jax import lax
from jax.experimental import pallas as pl
from jax.experimental.pallas import tpu as pltpu
from jax.experimental.pallas import tpu_sc as plsc

_NC = 2
_NS = 16
_LANES = 16


def _k1a_body(nf_ref, p_ref, q_ref, o_ref):
    o_ref[...] = (
        jnp.dot(nf_ref[...], p_ref[...], preferred_element_type=jnp.float32)
        + q_ref[...]
    )


def _k1b_body(ef_ref, mw_ref, o_ref):
    o_ref[...] = jnp.dot(ef_ref[...], mw_ref[...],
                         preferred_element_type=jnp.float32)


def _k2_body(a10_ref, a11_ref, a20_ref, a21_ref, nf_ref,
             A_ref, B_ref, Wc_ref, c0_ref, o_ref):
    accV = a10_ref[...] + a11_ref[...]
    s2 = a20_ref[...] + a21_ref[...]
    accE = s2[:, :16]
    dV = jnp.maximum(s2[:, 16:17], 1e-30)
    dE = jnp.maximum(s2[:, 17:18], 1e-30)
    deg = s2[:, 18:19]
    emb = (
        jnp.dot(accV / dV, A_ref[...], preferred_element_type=jnp.float32)
        + jnp.dot(accE / dE, B_ref[...], preferred_element_type=jnp.float32)
        + deg * (jnp.dot(nf_ref[...], Wc_ref[...],
                         preferred_element_type=jnp.float32) + c0_ref[...])
    )
    m = jnp.max(emb, axis=1, keepdims=True)
    ex = jnp.exp(emb - m)
    o_ref[...] = ex / jnp.sum(ex, axis=1, keepdims=True)


def _make_sc_scalar_kernel(N, E, DN, DE, NPAD):
    NT = _NC * _NS
    EP = E // NT
    C = 16
    NCHUNK = EP // C
    NP4 = NPAD // 4
    SR2 = NP4 // _NS

    mesh = plsc.VectorSubcoreMesh(core_axis_name="c", subcore_axis_name="s")

    NH = NCHUNK // 2

    def body(ef, u, v, s0, s1, s2, s3, ee, zst, evw, acc2,
             u_v, v_v, ee_v, s0_v, s1_v, s2_v, s3_v,
             ev_slab, pay2, ef_v, es0, es1, ps0, ps1, acc2_sh):
        cid = lax.axis_index("c")
        sid = lax.axis_index("s")
        base = (cid * _NS + sid) * EP
        efsems = (es0, es1)
        psems = (ps0, ps1)

        pltpu.sync_copy(zst.at[pl.ds(0, SR2)], acc2_sh.at[pl.ds(sid * SR2, SR2)])

        pltpu.sync_copy(u.at[pl.ds(base, EP)], u_v)
        pltpu.sync_copy(v.at[pl.ds(base, EP)], v_v)
        pltpu.sync_copy(ee.at[pl.ds(base, EP)], ee_v)
        pltpu.sync_copy(s0, s0_v)
        pltpu.sync_copy(s1, s1_v)
        pltpu.sync_copy(s2, s2_v)
        pltpu.sync_copy(s3, s3_v)

        zv = jnp.zeros((_LANES,), jnp.float32)
        for b in range(2):
            for j in range(C):
                for r in range(8):
                    pay2[b, j, pl.ds(r * _LANES, _LANES)] = zv

        plsc.subcore_barrier()

        row_iota = lax.iota(jnp.int32, _LANES)
        ones16 = jnp.ones((_LANES,), jnp.float32)

        for b in range(2):
            pltpu.async_copy(ef.at[pl.ds(base + b * C, C)],
                             ef_v.at[b], efsems[b])

        def pair(kk, carry):
            for b in range(2):
                off = (kk * 2 + b) * C
                u16 = u_v[pl.ds(off, _LANES)]
                v16 = v_v[pl.ds(off, _LANES)]
                sVu = plsc.load_gather(s0_v, [u16])
                sVv = plsc.load_gather(s1_v, [v16])
                sEu = plsc.load_gather(s2_v, [u16])
                sEv = plsc.load_gather(s3_v, [v16])
                ee16 = ee_v[pl.ds(off, _LANES)]
                aV = sVu + sVv
                aV = jnp.where(aV >= 0.0, aV, aV * 0.01)
                aE = sEu + sEv + ee16
                aE = jnp.where(aE >= 0.0, aE, aE * 0.01)
                eV = jnp.exp(aV)
                eEx = jnp.exp(aE)
                ev_slab[pl.ds(off, _LANES)] = eV

                gcol = lax.shift_left(jnp.bitwise_and(u16, 3), 5)

                pltpu.make_async_copy(ef.at[pl.ds(0, C)],
                                      ef_v.at[b], efsems[b]).wait()

                @pl.when(kk >= 1)
                def _():
                    pltpu.make_async_copy(acc2.at[0, pl.ds(0, C)],
                                          pay2.at[b], psems[b]).wait()
                    u16o = u_v[pl.ds(off - 2 * C, _LANES)]
                    gco = lax.shift_left(jnp.bitwise_and(u16o, 3), 5)
                    for f in range(DE + 3):
                        plsc.store_scatter(pay2.at[b],
                                           [row_iota, gco + f], zv)

                for f in range(DE):
                    colf = jnp.full((_LANES,), f, jnp.int32)
                    efc = plsc.load_gather(ef_v.at[b], [row_iota, colf])
                    plsc.store_scatter(pay2.at[b], [row_iota, gcol + f],
                                       efc * eEx)
                plsc.store_scatter(pay2.at[b], [row_iota, gcol + DE], eV)
                plsc.store_scatter(pay2.at[b], [row_iota, gcol + (DE + 1)], eEx)
                plsc.store_scatter(pay2.at[b], [row_iota, gcol + (DE + 2)], ones16)

                u4 = lax.shift_right_logical(u16, 2)
                pltpu.async_copy(pay2.at[b], acc2_sh.at[u4], psems[b],
                                 add=True)

                @pl.when(kk < NH - 1)
                def _():
                    pltpu.async_copy(ef.at[pl.ds(base + off + 2 * C, C)],
                                     ef_v.at[b], efsems[b])
            return carry

        lax.fori_loop(0, NH, pair, 0)

        for b in range(2):
            pltpu.make_async_copy(acc2.at[0, pl.ds(0, C)],
                                  pay2.at[b], psems[b]).wait()

        pltpu.sync_copy(ev_slab, evw.at[pl.ds(base, EP)])
        plsc.subcore_barrier()
        pltpu.sync_copy(acc2_sh.at[pl.ds(sid * SR2, SR2)],
                        acc2.at[cid, pl.ds(sid * SR2, SR2)])

    return functools.partial(
        pl.kernel,
        out_type=(
            jax.ShapeDtypeStruct((E,), jnp.float32),
            jax.ShapeDtypeStruct((_NC, NP4, 128), jnp.float32),
        ),
        mesh=mesh,
        compiler_params=pltpu.CompilerParams(needs_layout_passes=False),
        scratch_types=[
            pltpu.VMEM((EP,), jnp.int32),
            pltpu.VMEM((EP,), jnp.int32),
            pltpu.VMEM((EP,), jnp.float32),
            pltpu.VMEM((N,), jnp.float32),
            pltpu.VMEM((N,), jnp.float32),
            pltpu.VMEM((N,), jnp.float32),
            pltpu.VMEM((N,), jnp.float32),
            pltpu.VMEM((EP,), jnp.float32),
            pltpu.VMEM((2, C, 128), jnp.float32),
            pltpu.VMEM((2, C, DE), jnp.float32),
            pltpu.SemaphoreType.DMA,
            pltpu.SemaphoreType.DMA,
            pltpu.SemaphoreType.DMA,
            pltpu.SemaphoreType.DMA,
            pltpu.VMEM_SHARED((NP4, 128), jnp.float32),
        ],
    )(body)


def _make_sc_rows_kernel(N, E, DN, NPAD):
    NT = _NC * _NS
    EP = E // NT
    C = 16
    NCHUNK = EP // C
    SR1 = NPAD // _NS

    mesh = plsc.VectorSubcoreMesh(core_axis_name="c", subcore_axis_name="s")

    NH = NCHUNK // 2

    def body(nf, u, v, evw, zst, acc1,
             u_v, v_v, ev_slab, rows, pay, ev_v,
             gs0, gs1, ss0, ss1, acc1_sh):
        cid = lax.axis_index("c")
        sid = lax.axis_index("s")
        base = (cid * _NS + sid) * EP
        gsems = (gs0, gs1)
        ssems = (ss0, ss1)

        pltpu.sync_copy(zst, acc1_sh.at[pl.ds(sid * SR1, SR1)])
        pltpu.sync_copy(u.at[pl.ds(base, EP)], u_v)
        pltpu.sync_copy(v.at[pl.ds(base, EP)], v_v)
        pltpu.sync_copy(evw.at[pl.ds(base, EP)], ev_slab)
        plsc.subcore_barrier()

        for b in range(2):
            pltpu.async_copy(nf.at[v_v[pl.ds(b * C, _LANES)]],
                             rows.at[b], gsems[b])

        def pair(kk, carry):
            for b in range(2):
                off = (kk * 2 + b) * C
                u16 = u_v[pl.ds(off, _LANES)]
                eV = ev_slab[pl.ds(off, _LANES)]
                pltpu.make_async_copy(nf.at[pl.ds(0, C)],
                                      rows.at[b], gsems[b]).wait()

                @pl.when(kk >= 1)
                def _():
                    pltpu.make_async_copy(nf.at[pl.ds(0, C)],
                                          pay.at[b], ssems[b]).wait()

                ev_v[...] = eV
                for j in range(C):
                    sv = plsc.load_gather(
                        ev_v, [jnp.full((_LANES,), j, jnp.int32)])
                    for r in range(DN // _LANES):
                        pay[b, j, pl.ds(r * _LANES, _LANES)] = (
                            rows[b, j, pl.ds(r * _LANES, _LANES)] * sv)
                pltpu.async_copy(pay.at[b], acc1_sh.at[u16], ssems[b],
                                 add=True)

                @pl.when(kk < NH - 1)
                def _():
                    v16n = v_v[pl.ds(off + 2 * C, _LANES)]
                    pltpu.async_copy(nf.at[v16n], rows.at[b], gsems[b])
            return carry

        lax.fori_loop(0, NH, pair, 0)

        for b in range(2):
            pltpu.make_async_copy(nf.at[pl.ds(0, C)],
                                  pay.at[b], ssems[b]).wait()

        plsc.subcore_barrier()
        pltpu.sync_copy(acc1_sh.at[pl.ds(sid * SR1, SR1)],
                        acc1.at[cid, pl.ds(sid * SR1, SR1)])

    return functools.partial(
        pl.kernel,
        out_type=jax.ShapeDtypeStruct((_NC, NPAD, 128), jnp.float32),
        mesh=mesh,
        compiler_params=pltpu.CompilerParams(needs_layout_passes=False),
        scratch_types=[
            pltpu.VMEM((EP,), jnp.int32),
            pltpu.VMEM((EP,), jnp.int32),
            pltpu.VMEM((EP,), jnp.float32),
            pltpu.VMEM((2, C, DN), jnp.float32),
            pltpu.VMEM((2, C, DN), jnp.float32),
            pltpu.VMEM((_LANES,), jnp.float32),
            pltpu.SemaphoreType.DMA,
            pltpu.SemaphoreType.DMA,
            pltpu.SemaphoreType.DMA,
            pltpu.SemaphoreType.DMA,
            pltpu.VMEM_SHARED((NPAD, 128), jnp.float32),
        ],
    )(body)


def kernel(node_features, edge_features, edge_indexes,
           W1, b1, W2, b2, W3, b3, Wv, bv, We, be):
    N, DN = node_features.shape
    E, DE = edge_features.shape
    DO = W3.shape[1]
    u = edge_indexes[0]
    v = edge_indexes[1]
    NPAD = -(-N // 256) * 256

    C4 = jnp.stack([Wv[:DN, 0], Wv[DN:, 0], We[:DN, 0], We[DN:2 * DN, 0]],
                   axis=1)
    P = W1 @ C4
    q = b1 @ C4
    q = q.at[0].add(bv[0])
    q = q.at[2].add(b2 @ We[2 * DN:, 0] + be[0])
    P_pad = jnp.zeros((DN, 128), jnp.float32).at[:, :4].set(P)
    q_pad = jnp.zeros((1, 128), jnp.float32).at[0, :4].set(q)
    w2e = W2 @ We[2 * DN:, 0]
    A = W1 @ W3[:DN]
    B = W2 @ W3[DN:DN + DE]
    W3c = W3[DN + DE:]
    c0 = (b1 @ W3[:DN] + b2 @ W3[DN:DN + DE] + b3)[None, :]

    S_full = pl.pallas_call(
        _k1a_body,
        out_shape=jax.ShapeDtypeStruct((N, 128), jnp.float32),
    )(node_features, P_pad, q_pad)
    s0, s1, s2, s3 = (S_full[:, 0], S_full[:, 1], S_full[:, 2], S_full[:, 3])

    gsel = jnp.arange(128) // DE
    Mw = jnp.where(gsel[:, None] == jnp.arange(8)[None, :],
                   jnp.tile(w2e, 8)[:, None], 0.0)
    ef2 = edge_features.reshape(E // 8, 128)
    eE = pl.pallas_call(
        _k1b_body,
        out_shape=jax.ShapeDtypeStruct((E // 8, 8), jnp.float32),
    )(ef2, Mw).reshape(E)

    zstripe = jnp.zeros((NPAD // _NS, 128), jnp.float32)
    evw, acc2 = _make_sc_scalar_kernel(N, E, DN, DE, NPAD)(
        edge_features, u, v, s0, s1, s2, s3, eE, zstripe)
    acc1 = _make_sc_rows_kernel(N, E, DN, NPAD)(
        node_features, u, v, evw, zstripe)
    acc2r = acc2.reshape(_NC, NPAD, 32)

    nf_pad = jnp.pad(node_features, ((0, NPAD - N), (0, 0)))
    out = pl.pallas_call(
        _k2_body,
        out_shape=jax.ShapeDtypeStruct((NPAD, DO), jnp.float32),
    )(acc1[0], acc1[1], acc2r[0], acc2r[1], nf_pad, A, B, W3c, c0)
    return out[:N]


# speedup vs baseline: 18.1814x; 1.2053x over previous
import functools

import jax
import jax.numpy as jnp
from jax import lax
from jax.experimental import pallas as pl
from jax.experimental.pallas import tpu as pltpu
from jax.experimental.pallas import tpu_sc as plsc

_NC = 2
_NS = 16
_LANES = 16


def _k1a_body(nf_ref, p_ref, q_ref, o_ref):
    o_ref[...] = (
        jnp.dot(nf_ref[...], p_ref[...], preferred_element_type=jnp.float32)
        + q_ref[...]
    )


def _k1b_body(ef_ref, mw_ref, o_ref):
    o_ref[...] = jnp.dot(ef_ref[...], mw_ref[...],
                         preferred_element_type=jnp.float32)


def _k2_body(a10_ref, a11_ref, a20_ref, a21_ref, nf_ref,
             A_ref, B_ref, Wc_ref, c0_ref, o_ref):
    accV = a10_ref[...] + a11_ref[...]
    s2 = a20_ref[...] + a21_ref[...]
    accE = s2[:, :16]
    dV = jnp.maximum(s2[:, 16:17], 1e-30)
    dE = jnp.maximum(s2[:, 17:18], 1e-30)
    deg = s2[:, 18:19]
    emb = (
        jnp.dot(accV / dV, A_ref[...], preferred_element_type=jnp.float32)
        + jnp.dot(accE / dE, B_ref[...], preferred_element_type=jnp.float32)
        + deg * (jnp.dot(nf_ref[...], Wc_ref[...],
                         preferred_element_type=jnp.float32) + c0_ref[...])
    )
    m = jnp.max(emb, axis=1, keepdims=True)
    ex = jnp.exp(emb - m)
    o_ref[...] = ex / jnp.sum(ex, axis=1, keepdims=True)


def _make_sc_scalar_kernel(N, E, DN, DE, NPAD):
    NT = _NC * _NS
    EP = E // NT
    C = 16
    NCHUNK = EP // C
    NP4 = NPAD // 4
    SR2 = NP4 // _NS

    mesh = plsc.VectorSubcoreMesh(core_axis_name="c", subcore_axis_name="s")

    NB = 4
    NH = NCHUNK // NB

    def body(ef, u, v, s0, s1, s2, s3, ee, zst, evw, acc2,
             u_v, v_v, ee_v, s0_v, s1_v, s2_v, s3_v,
             ev_slab, pay2, ef_v, es0, es1, es2, es3,
             ps0, ps1, ps2, ps3, acc2_sh):
        cid = lax.axis_index("c")
        sid = lax.axis_index("s")
        base = (cid * _NS + sid) * EP
        efsems = (es0, es1, es2, es3)
        psems = (ps0, ps1, ps2, ps3)

        pltpu.sync_copy(zst.at[pl.ds(0, SR2)], acc2_sh.at[pl.ds(sid * SR2, SR2)])

        pltpu.sync_copy(u.at[pl.ds(base, EP)], u_v)
        pltpu.sync_copy(v.at[pl.ds(base, EP)], v_v)
        pltpu.sync_copy(ee.at[pl.ds(base, EP)], ee_v)
        pltpu.sync_copy(s0, s0_v)
        pltpu.sync_copy(s1, s1_v)
        pltpu.sync_copy(s2, s2_v)
        pltpu.sync_copy(s3, s3_v)

        zv = jnp.zeros((_LANES,), jnp.float32)
        for b in range(4):
            for j in range(C):
                for r in range(8):
                    pay2[b, j, pl.ds(r * _LANES, _LANES)] = zv

        plsc.subcore_barrier()

        row_iota = lax.iota(jnp.int32, _LANES)
        ones16 = jnp.ones((_LANES,), jnp.float32)

        for b in range(4):
            pltpu.async_copy(ef.at[pl.ds(base + b * C, C)],
                             ef_v.at[b], efsems[b])

        def pair(kk, carry):
            for b in range(4):
                off = (kk * 4 + b) * C
                u16 = u_v[pl.ds(off, _LANES)]
                v16 = v_v[pl.ds(off, _LANES)]
                sVu = plsc.load_gather(s0_v, [u16])
                sVv = plsc.load_gather(s1_v, [v16])
                sEu = plsc.load_gather(s2_v, [u16])
                sEv = plsc.load_gather(s3_v, [v16])
                ee16 = ee_v[pl.ds(off, _LANES)]
                aV = sVu + sVv
                aV = jnp.where(aV >= 0.0, aV, aV * 0.01)
                aE = sEu + sEv + ee16
                aE = jnp.where(aE >= 0.0, aE, aE * 0.01)
                eV = jnp.exp(aV)
                eEx = jnp.exp(aE)
                ev_slab[pl.ds(off, _LANES)] = eV

                gcol = lax.shift_left(jnp.bitwise_and(u16, 3), 5)

                pltpu.make_async_copy(ef.at[pl.ds(0, C)],
                                      ef_v.at[b], efsems[b]).wait()

                @pl.when(kk >= 1)
                def _():
                    pltpu.make_async_copy(acc2.at[0, pl.ds(0, C)],
                                          pay2.at[b], psems[b]).wait()
                    u16o = u_v[pl.ds(off - 4 * C, _LANES)]
                    gco = lax.shift_left(jnp.bitwise_and(u16o, 3), 5)
                    for f in range(DE + 3):
                        plsc.store_scatter(pay2.at[b],
                                           [row_iota, gco + f], zv)

                for f in range(DE):
                    colf = jnp.full((_LANES,), f, jnp.int32)
                    efc = plsc.load_gather(ef_v.at[b], [row_iota, colf])
                    plsc.store_scatter(pay2.at[b], [row_iota, gcol + f],
                                       efc * eEx)
                plsc.store_scatter(pay2.at[b], [row_iota, gcol + DE], eV)
                plsc.store_scatter(pay2.at[b], [row_iota, gcol + (DE + 1)], eEx)
                plsc.store_scatter(pay2.at[b], [row_iota, gcol + (DE + 2)], ones16)

                u4 = lax.shift_right_logical(u16, 2)
                pltpu.async_copy(pay2.at[b], acc2_sh.at[u4], psems[b],
                                 add=True)

                @pl.when(kk < NH - 1)
                def _():
                    pltpu.async_copy(ef.at[pl.ds(base + off + 4 * C, C)],
                                     ef_v.at[b], efsems[b])
            return carry

        lax.fori_loop(0, NH, pair, 0)

        for b in range(4):
            pltpu.make_async_copy(acc2.at[0, pl.ds(0, C)],
                                  pay2.at[b], psems[b]).wait()

        for k in range(NH * NB, NCHUNK):
            off = k * C
            u16 = u_v[pl.ds(off, _LANES)]
            v16 = v_v[pl.ds(off, _LANES)]
            sVu = plsc.load_gather(s0_v, [u16])
            sVv = plsc.load_gather(s1_v, [v16])
            sEu = plsc.load_gather(s2_v, [u16])
            sEv = plsc.load_gather(s3_v, [v16])
            ee16 = ee_v[pl.ds(off, _LANES)]
            aV = sVu + sVv
            aV = jnp.where(aV >= 0.0, aV, aV * 0.01)
            aE = sEu + sEv + ee16
            aE = jnp.where(aE >= 0.0, aE, aE * 0.01)
            eV = jnp.exp(aV)
            eEx = jnp.exp(aE)
            ev_slab[pl.ds(off, _LANES)] = eV
            b = k % NB
            u16o = u_v[pl.ds((k - NB) * C, _LANES)]
            gco = lax.shift_left(jnp.bitwise_and(u16o, 3), 5)
            for f in range(DE + 3):
                plsc.store_scatter(pay2.at[b], [row_iota, gco + f], zv)
            pltpu.sync_copy(ef.at[pl.ds(base + off, C)], ef_v.at[b])
            gcol = lax.shift_left(jnp.bitwise_and(u16, 3), 5)
            for f in range(DE):
                colf = jnp.full((_LANES,), f, jnp.int32)
                efc = plsc.load_gather(ef_v.at[b], [row_iota, colf])
                plsc.store_scatter(pay2.at[b], [row_iota, gcol + f],
                                   efc * eEx)
            plsc.store_scatter(pay2.at[b], [row_iota, gcol + DE], eV)
            plsc.store_scatter(pay2.at[b], [row_iota, gcol + (DE + 1)], eEx)
            plsc.store_scatter(pay2.at[b], [row_iota, gcol + (DE + 2)], ones16)
            u4 = lax.shift_right_logical(u16, 2)
            pltpu.sync_copy(pay2.at[b], acc2_sh.at[u4], add=True)

        pltpu.sync_copy(ev_slab, evw.at[pl.ds(base, EP)])
        plsc.subcore_barrier()
        pltpu.sync_copy(acc2_sh.at[pl.ds(sid * SR2, SR2)],
                        acc2.at[cid, pl.ds(sid * SR2, SR2)])

    return functools.partial(
        pl.kernel,
        out_type=(
            jax.ShapeDtypeStruct((E,), jnp.float32),
            jax.ShapeDtypeStruct((_NC, NP4, 128), jnp.float32),
        ),
        mesh=mesh,
        compiler_params=pltpu.CompilerParams(needs_layout_passes=False),
        scratch_types=[
            pltpu.VMEM((EP,), jnp.int32),
            pltpu.VMEM((EP,), jnp.int32),
            pltpu.VMEM((EP,), jnp.float32),
            pltpu.VMEM((N,), jnp.float32),
            pltpu.VMEM((N,), jnp.float32),
            pltpu.VMEM((N,), jnp.float32),
            pltpu.VMEM((N,), jnp.float32),
            pltpu.VMEM((EP,), jnp.float32),
            pltpu.VMEM((4, C, 128), jnp.float32),
            pltpu.VMEM((4, C, DE), jnp.float32),
            pltpu.SemaphoreType.DMA,
            pltpu.SemaphoreType.DMA,
            pltpu.SemaphoreType.DMA,
            pltpu.SemaphoreType.DMA,
            pltpu.SemaphoreType.DMA,
            pltpu.SemaphoreType.DMA,
            pltpu.SemaphoreType.DMA,
            pltpu.SemaphoreType.DMA,
            pltpu.VMEM_SHARED((NP4, 128), jnp.float32),
        ],
    )(body)


def _make_sc_rows_kernel(N, E, DN, NPAD):
    NT = _NC * _NS
    EP = E // NT
    C = 16
    NCHUNK = EP // C
    SR1 = NPAD // _NS

    mesh = plsc.VectorSubcoreMesh(core_axis_name="c", subcore_axis_name="s")

    NB = 4
    NH = NCHUNK // NB

    def body(nf, u, v, evw, zst, acc1,
             u_v, v_v, ev_slab, rows, pay, ev_v,
             gs0, gs1, gs2, gs3, ss0, ss1, ss2, ss3, acc1_sh):
        cid = lax.axis_index("c")
        sid = lax.axis_index("s")
        base = (cid * _NS + sid) * EP
        gsems = (gs0, gs1, gs2, gs3)
        ssems = (ss0, ss1, ss2, ss3)

        pltpu.sync_copy(zst, acc1_sh.at[pl.ds(sid * SR1, SR1)])
        pltpu.sync_copy(u.at[pl.ds(base, EP)], u_v)
        pltpu.sync_copy(v.at[pl.ds(base, EP)], v_v)
        pltpu.sync_copy(evw.at[pl.ds(base, EP)], ev_slab)
        plsc.subcore_barrier()

        for b in range(4):
            pltpu.async_copy(nf.at[v_v[pl.ds(b * C, _LANES)]],
                             rows.at[b], gsems[b])

        def pair(kk, carry):
            for b in range(4):
                off = (kk * 4 + b) * C
                u16 = u_v[pl.ds(off, _LANES)]
                eV = ev_slab[pl.ds(off, _LANES)]
                pltpu.make_async_copy(nf.at[pl.ds(0, C)],
                                      rows.at[b], gsems[b]).wait()

                @pl.when(kk >= 1)
                def _():
                    pltpu.make_async_copy(nf.at[pl.ds(0, C)],
                                          pay.at[b], ssems[b]).wait()

                ev_v[...] = eV
                for j in range(C):
                    sv = plsc.load_gather(
                        ev_v, [jnp.full((_LANES,), j, jnp.int32)])
                    for r in range(DN // _LANES):
                        pay[b, j, pl.ds(r * _LANES, _LANES)] = (
                            rows[b, j, pl.ds(r * _LANES, _LANES)] * sv)
                pltpu.async_copy(pay.at[b], acc1_sh.at[u16], ssems[b],
                                 add=True)

                @pl.when(kk < NH - 1)
                def _():
                    v16n = v_v[pl.ds(off + 4 * C, _LANES)]
                    pltpu.async_copy(nf.at[v16n], rows.at[b], gsems[b])
            return carry

        lax.fori_loop(0, NH, pair, 0)

        for b in range(4):
            pltpu.make_async_copy(nf.at[pl.ds(0, C)],
                                  pay.at[b], ssems[b]).wait()

        for k in range(NH * NB, NCHUNK):
            off = k * C
            u16 = u_v[pl.ds(off, _LANES)]
            v16 = v_v[pl.ds(off, _LANES)]
            eV = ev_slab[pl.ds(off, _LANES)]
            b = k % NB
            pltpu.sync_copy(nf.at[v16], rows.at[b])
            ev_v[...] = eV
            for j in range(C):
                sv = plsc.load_gather(
                    ev_v, [jnp.full((_LANES,), j, jnp.int32)])
                for r in range(DN // _LANES):
                    pay[b, j, pl.ds(r * _LANES, _LANES)] = (
                        rows[b, j, pl.ds(r * _LANES, _LANES)] * sv)
            pltpu.sync_copy(pay.at[b], acc1_sh.at[u16], add=True)

        plsc.subcore_barrier()
        pltpu.sync_copy(acc1_sh.at[pl.ds(sid * SR1, SR1)],
                        acc1.at[cid, pl.ds(sid * SR1, SR1)])

    return functools.partial(
        pl.kernel,
        out_type=jax.ShapeDtypeStruct((_NC, NPAD, 128), jnp.float32),
        mesh=mesh,
        compiler_params=pltpu.CompilerParams(needs_layout_passes=False),
        scratch_types=[
            pltpu.VMEM((EP,), jnp.int32),
            pltpu.VMEM((EP,), jnp.int32),
            pltpu.VMEM((EP,), jnp.float32),
            pltpu.VMEM((4, C, DN), jnp.float32),
            pltpu.VMEM((4, C, DN), jnp.float32),
            pltpu.VMEM((_LANES,), jnp.float32),
            pltpu.SemaphoreType.DMA,
            pltpu.SemaphoreType.DMA,
            pltpu.SemaphoreType.DMA,
            pltpu.SemaphoreType.DMA,
            pltpu.SemaphoreType.DMA,
            pltpu.SemaphoreType.DMA,
            pltpu.SemaphoreType.DMA,
            pltpu.SemaphoreType.DMA,
            pltpu.VMEM_SHARED((NPAD, 128), jnp.float32),
        ],
    )(body)


def kernel(node_features, edge_features, edge_indexes,
           W1, b1, W2, b2, W3, b3, Wv, bv, We, be):
    N, DN = node_features.shape
    E, DE = edge_features.shape
    DO = W3.shape[1]
    u = edge_indexes[0]
    v = edge_indexes[1]
    NPAD = -(-N // 256) * 256

    C4 = jnp.stack([Wv[:DN, 0], Wv[DN:, 0], We[:DN, 0], We[DN:2 * DN, 0]],
                   axis=1)
    P = W1 @ C4
    q = b1 @ C4
    q = q.at[0].add(bv[0])
    q = q.at[2].add(b2 @ We[2 * DN:, 0] + be[0])
    P_pad = jnp.zeros((DN, 128), jnp.float32).at[:, :4].set(P)
    q_pad = jnp.zeros((1, 128), jnp.float32).at[0, :4].set(q)
    w2e = W2 @ We[2 * DN:, 0]
    A = W1 @ W3[:DN]
    B = W2 @ W3[DN:DN + DE]
    W3c = W3[DN + DE:]
    c0 = (b1 @ W3[:DN] + b2 @ W3[DN:DN + DE] + b3)[None, :]

    S_full = pl.pallas_call(
        _k1a_body,
        out_shape=jax.ShapeDtypeStruct((N, 128), jnp.float32),
    )(node_features, P_pad, q_pad)
    s0, s1, s2, s3 = (S_full[:, 0], S_full[:, 1], S_full[:, 2], S_full[:, 3])

    gsel = jnp.arange(128) // DE
    Mw = jnp.where(gsel[:, None] == jnp.arange(8)[None, :],
                   jnp.tile(w2e, 8)[:, None], 0.0)
    ef2 = edge_features.reshape(E // 8, 128)
    eE = pl.pallas_call(
        _k1b_body,
        out_shape=jax.ShapeDtypeStruct((E // 8, 8), jnp.float32),
    )(ef2, Mw).reshape(E)

    zstripe = jnp.zeros((NPAD // _NS, 128), jnp.float32)
    evw, acc2 = _make_sc_scalar_kernel(N, E, DN, DE, NPAD)(
        edge_features, u, v, s0, s1, s2, s3, eE, zstripe)
    acc1 = _make_sc_rows_kernel(N, E, DN, NPAD)(
        node_features, u, v, evw, zstripe)
    acc2r = acc2.reshape(_NC, NPAD, 32)

    nf_pad = jnp.pad(node_features, ((0, NPAD - N), (0, 0)))
    out = pl.pallas_call(
        _k2_body,
        out_shape=jax.ShapeDtypeStruct((NPAD, DO), jnp.float32),
    )(acc1[0], acc1[1], acc2r[0], acc2r[1], nf_pad, A, B, W3c, c0)
    return out[:N]


# speedup vs baseline: 18.3744x; 1.0106x over previous
import functools

import jax
import jax.numpy as jnp
from jax import lax
from jax.experimental import pallas as pl
from jax.experimental.pallas import tpu as pltpu
from jax.experimental.pallas import tpu_sc as plsc

_NC = 2
_NS = 16
_LANES = 16


def _k1_body(nf_ref, p_ref, q_ref, ef_ref, mw_ref, s_ref, e_ref):
    s_ref[...] = (
        jnp.dot(nf_ref[...], p_ref[...], preferred_element_type=jnp.float32)
        + q_ref[...]
    )
    e_ref[...] = jnp.dot(ef_ref[...], mw_ref[...],
                         preferred_element_type=jnp.float32)


def _k2_body(a10_ref, a11_ref, a20_ref, a21_ref, nf_ref,
             A_ref, B_ref, Wc_ref, c0_ref, o_ref):
    n = nf_ref.shape[0]
    accV = a10_ref[:n, :] + a11_ref[:n, :]
    s2 = a20_ref[:n, :] + a21_ref[:n, :]
    accE = s2[:, :16]
    dV = jnp.maximum(s2[:, 16:17], 1e-30)
    dE = jnp.maximum(s2[:, 17:18], 1e-30)
    deg = s2[:, 18:19]
    emb = (
        jnp.dot(accV / dV, A_ref[...], preferred_element_type=jnp.float32)
        + jnp.dot(accE / dE, B_ref[...], preferred_element_type=jnp.float32)
        + deg * (jnp.dot(nf_ref[...], Wc_ref[...],
                         preferred_element_type=jnp.float32) + c0_ref[...])
    )
    m = jnp.max(emb, axis=1, keepdims=True)
    ex = jnp.exp(emb - m)
    o_ref[...] = ex / jnp.sum(ex, axis=1, keepdims=True)


def _make_sc_scalar_kernel(N, E, DN, DE, NPAD):
    NT = _NC * _NS
    EP = E // NT
    C = 16
    NCHUNK = EP // C
    NP4 = NPAD // 4
    SR2 = NP4 // _NS

    mesh = plsc.VectorSubcoreMesh(core_axis_name="c", subcore_axis_name="s")

    NB = 4
    NH = NCHUNK // NB

    def body(ef, u, v, s0, s1, s2, s3, ee, zst, evw, acc2,
             u_v, v_v, ee_v, s0_v, s1_v, s2_v, s3_v,
             ev_slab, pay2, ef_v, es0, es1, es2, es3,
             ps0, ps1, ps2, ps3, acc2_sh):
        cid = lax.axis_index("c")
        sid = lax.axis_index("s")
        base = (cid * _NS + sid) * EP
        efsems = (es0, es1, es2, es3)
        psems = (ps0, ps1, ps2, ps3)

        pltpu.sync_copy(zst.at[pl.ds(0, SR2)], acc2_sh.at[pl.ds(sid * SR2, SR2)])

        pltpu.sync_copy(u.at[pl.ds(base, EP)], u_v)
        pltpu.sync_copy(v.at[pl.ds(base, EP)], v_v)
        pltpu.sync_copy(ee.at[pl.ds(base, EP)], ee_v)
        pltpu.sync_copy(s0, s0_v)
        pltpu.sync_copy(s1, s1_v)
        pltpu.sync_copy(s2, s2_v)
        pltpu.sync_copy(s3, s3_v)

        zv = jnp.zeros((_LANES,), jnp.float32)
        for b in range(4):
            for j in range(C):
                for r in range(8):
                    pay2[b, j, pl.ds(r * _LANES, _LANES)] = zv

        plsc.subcore_barrier()

        row_iota = lax.iota(jnp.int32, _LANES)
        ones16 = jnp.ones((_LANES,), jnp.float32)

        for b in range(4):
            pltpu.async_copy(ef.at[pl.ds(base + b * C, C)],
                             ef_v.at[b], efsems[b])

        def pair(kk, carry):
            for b in range(4):
                off = (kk * 4 + b) * C
                u16 = u_v[pl.ds(off, _LANES)]
                v16 = v_v[pl.ds(off, _LANES)]
                sVu = plsc.load_gather(s0_v, [u16])
                sVv = plsc.load_gather(s1_v, [v16])
                sEu = plsc.load_gather(s2_v, [u16])
                sEv = plsc.load_gather(s3_v, [v16])
                ee16 = ee_v[pl.ds(off, _LANES)]
                aV = sVu + sVv
                aV = jnp.where(aV >= 0.0, aV, aV * 0.01)
                aE = sEu + sEv + ee16
                aE = jnp.where(aE >= 0.0, aE, aE * 0.01)
                eV = jnp.exp(aV)
                eEx = jnp.exp(aE)
                ev_slab[pl.ds(off, _LANES)] = eV

                gcol = lax.shift_left(jnp.bitwise_and(u16, 3), 5)

                pltpu.make_async_copy(ef.at[pl.ds(0, C)],
                                      ef_v.at[b], efsems[b]).wait()

                @pl.when(kk >= 1)
                def _():
                    pltpu.make_async_copy(acc2.at[0, pl.ds(0, C)],
                                          pay2.at[b], psems[b]).wait()
                    u16o = u_v[pl.ds(off - 4 * C, _LANES)]
                    gco = lax.shift_left(jnp.bitwise_and(u16o, 3), 5)
                    for f in range(DE + 3):
                        plsc.store_scatter(pay2.at[b],
                                           [row_iota, gco + f], zv)

                for f in range(DE):
                    colf = jnp.full((_LANES,), f, jnp.int32)
                    efc = plsc.load_gather(ef_v.at[b], [row_iota, colf])
                    plsc.store_scatter(pay2.at[b], [row_iota, gcol + f],
                                       efc * eEx)
                plsc.store_scatter(pay2.at[b], [row_iota, gcol + DE], eV)
                plsc.store_scatter(pay2.at[b], [row_iota, gcol + (DE + 1)], eEx)
                plsc.store_scatter(pay2.at[b], [row_iota, gcol + (DE + 2)], ones16)

                u4 = lax.shift_right_logical(u16, 2)
                pltpu.async_copy(pay2.at[b], acc2_sh.at[u4], psems[b],
                                 add=True)

                @pl.when(kk < NH - 1)
                def _():
                    pltpu.async_copy(ef.at[pl.ds(base + off + 4 * C, C)],
                                     ef_v.at[b], efsems[b])
            return carry

        lax.fori_loop(0, NH, pair, 0)

        for b in range(4):
            pltpu.make_async_copy(acc2.at[0, pl.ds(0, C)],
                                  pay2.at[b], psems[b]).wait()

        for k in range(NH * NB, NCHUNK):
            off = k * C
            u16 = u_v[pl.ds(off, _LANES)]
            v16 = v_v[pl.ds(off, _LANES)]
            sVu = plsc.load_gather(s0_v, [u16])
            sVv = plsc.load_gather(s1_v, [v16])
            sEu = plsc.load_gather(s2_v, [u16])
            sEv = plsc.load_gather(s3_v, [v16])
            ee16 = ee_v[pl.ds(off, _LANES)]
            aV = sVu + sVv
            aV = jnp.where(aV >= 0.0, aV, aV * 0.01)
            aE = sEu + sEv + ee16
            aE = jnp.where(aE >= 0.0, aE, aE * 0.01)
            eV = jnp.exp(aV)
            eEx = jnp.exp(aE)
            ev_slab[pl.ds(off, _LANES)] = eV
            b = k % NB
            u16o = u_v[pl.ds((k - NB) * C, _LANES)]
            gco = lax.shift_left(jnp.bitwise_and(u16o, 3), 5)
            for f in range(DE + 3):
                plsc.store_scatter(pay2.at[b], [row_iota, gco + f], zv)
            pltpu.sync_copy(ef.at[pl.ds(base + off, C)], ef_v.at[b])
            gcol = lax.shift_left(jnp.bitwise_and(u16, 3), 5)
            for f in range(DE):
                colf = jnp.full((_LANES,), f, jnp.int32)
                efc = plsc.load_gather(ef_v.at[b], [row_iota, colf])
                plsc.store_scatter(pay2.at[b], [row_iota, gcol + f],
                                   efc * eEx)
            plsc.store_scatter(pay2.at[b], [row_iota, gcol + DE], eV)
            plsc.store_scatter(pay2.at[b], [row_iota, gcol + (DE + 1)], eEx)
            plsc.store_scatter(pay2.at[b], [row_iota, gcol + (DE + 2)], ones16)
            u4 = lax.shift_right_logical(u16, 2)
            pltpu.sync_copy(pay2.at[b], acc2_sh.at[u4], add=True)

        pltpu.sync_copy(ev_slab, evw.at[pl.ds(base, EP)])
        plsc.subcore_barrier()
        pltpu.sync_copy(acc2_sh.at[pl.ds(sid * SR2, SR2)],
                        acc2.at[cid, pl.ds(sid * SR2, SR2)])

    return functools.partial(
        pl.kernel,
        out_type=(
            jax.ShapeDtypeStruct((E,), jnp.float32),
            jax.ShapeDtypeStruct((_NC, NP4, 128), jnp.float32),
        ),
        mesh=mesh,
        compiler_params=pltpu.CompilerParams(needs_layout_passes=False),
        scratch_types=[
            pltpu.VMEM((EP,), jnp.int32),
            pltpu.VMEM((EP,), jnp.int32),
            pltpu.VMEM((EP,), jnp.float32),
            pltpu.VMEM((N,), jnp.float32),
            pltpu.VMEM((N,), jnp.float32),
            pltpu.VMEM((N,), jnp.float32),
            pltpu.VMEM((N,), jnp.float32),
            pltpu.VMEM((EP,), jnp.float32),
            pltpu.VMEM((4, C, 128), jnp.float32),
            pltpu.VMEM((4, C, DE), jnp.float32),
            pltpu.SemaphoreType.DMA,
            pltpu.SemaphoreType.DMA,
            pltpu.SemaphoreType.DMA,
            pltpu.SemaphoreType.DMA,
            pltpu.SemaphoreType.DMA,
            pltpu.SemaphoreType.DMA,
            pltpu.SemaphoreType.DMA,
            pltpu.SemaphoreType.DMA,
            pltpu.VMEM_SHARED((NP4, 128), jnp.float32),
        ],
    )(body)


def _make_sc_rows_kernel(N, E, DN, NPAD):
    NT = _NC * _NS
    EP = E // NT
    C = 16
    NCHUNK = EP // C
    SR1 = NPAD // _NS

    mesh = plsc.VectorSubcoreMesh(core_axis_name="c", subcore_axis_name="s")

    NB = 4
    NH = NCHUNK // NB

    def body(nf, u, v, evw, zst, acc1,
             u_v, v_v, ev_slab, rows, pay, ev_v,
             gs0, gs1, gs2, gs3, ss0, ss1, ss2, ss3, acc1_sh):
        cid = lax.axis_index("c")
        sid = lax.axis_index("s")
        base = (cid * _NS + sid) * EP
        gsems = (gs0, gs1, gs2, gs3)
        ssems = (ss0, ss1, ss2, ss3)

        pltpu.sync_copy(zst, acc1_sh.at[pl.ds(sid * SR1, SR1)])
        pltpu.sync_copy(u.at[pl.ds(base, EP)], u_v)
        pltpu.sync_copy(v.at[pl.ds(base, EP)], v_v)
        pltpu.sync_copy(evw.at[pl.ds(base, EP)], ev_slab)
        plsc.subcore_barrier()

        for b in range(4):
            pltpu.async_copy(nf.at[v_v[pl.ds(b * C, _LANES)]],
                             rows.at[b], gsems[b])

        def pair(kk, carry):
            for b in range(4):
                off = (kk * 4 + b) * C
                u16 = u_v[pl.ds(off, _LANES)]
                eV = ev_slab[pl.ds(off, _LANES)]
                pltpu.make_async_copy(nf.at[pl.ds(0, C)],
                                      rows.at[b], gsems[b]).wait()

                @pl.when(kk >= 1)
                def _():
                    pltpu.make_async_copy(nf.at[pl.ds(0, C)],
                                          pay.at[b], ssems[b]).wait()

                ev_v[...] = eV
                for j in range(C):
                    sv = plsc.load_gather(
                        ev_v, [jnp.full((_LANES,), j, jnp.int32)])
                    for r in range(DN // _LANES):
                        pay[b, j, pl.ds(r * _LANES, _LANES)] = (
                            rows[b, j, pl.ds(r * _LANES, _LANES)] * sv)
                pltpu.async_copy(pay.at[b], acc1_sh.at[u16], ssems[b],
                                 add=True)

                @pl.when(kk < NH - 1)
                def _():
                    v16n = v_v[pl.ds(off + 4 * C, _LANES)]
                    pltpu.async_copy(nf.at[v16n], rows.at[b], gsems[b])
            return carry

        lax.fori_loop(0, NH, pair, 0)

        for b in range(4):
            pltpu.make_async_copy(nf.at[pl.ds(0, C)],
                                  pay.at[b], ssems[b]).wait()

        for k in range(NH * NB, NCHUNK):
            off = k * C
            u16 = u_v[pl.ds(off, _LANES)]
            v16 = v_v[pl.ds(off, _LANES)]
            eV = ev_slab[pl.ds(off, _LANES)]
            b = k % NB
            pltpu.sync_copy(nf.at[v16], rows.at[b])
            ev_v[...] = eV
            for j in range(C):
                sv = plsc.load_gather(
                    ev_v, [jnp.full((_LANES,), j, jnp.int32)])
                for r in range(DN // _LANES):
                    pay[b, j, pl.ds(r * _LANES, _LANES)] = (
                        rows[b, j, pl.ds(r * _LANES, _LANES)] * sv)
            pltpu.sync_copy(pay.at[b], acc1_sh.at[u16], add=True)

        plsc.subcore_barrier()
        pltpu.sync_copy(acc1_sh.at[pl.ds(sid * SR1, SR1)],
                        acc1.at[cid, pl.ds(sid * SR1, SR1)])

    return functools.partial(
        pl.kernel,
        out_type=jax.ShapeDtypeStruct((_NC, NPAD, 128), jnp.float32),
        mesh=mesh,
        compiler_params=pltpu.CompilerParams(needs_layout_passes=False),
        scratch_types=[
            pltpu.VMEM((EP,), jnp.int32),
            pltpu.VMEM((EP,), jnp.int32),
            pltpu.VMEM((EP,), jnp.float32),
            pltpu.VMEM((4, C, DN), jnp.float32),
            pltpu.VMEM((4, C, DN), jnp.float32),
            pltpu.VMEM((_LANES,), jnp.float32),
            pltpu.SemaphoreType.DMA,
            pltpu.SemaphoreType.DMA,
            pltpu.SemaphoreType.DMA,
            pltpu.SemaphoreType.DMA,
            pltpu.SemaphoreType.DMA,
            pltpu.SemaphoreType.DMA,
            pltpu.SemaphoreType.DMA,
            pltpu.SemaphoreType.DMA,
            pltpu.VMEM_SHARED((NPAD, 128), jnp.float32),
        ],
    )(body)


def kernel(node_features, edge_features, edge_indexes,
           W1, b1, W2, b2, W3, b3, Wv, bv, We, be):
    N, DN = node_features.shape
    E, DE = edge_features.shape
    DO = W3.shape[1]
    u = edge_indexes[0]
    v = edge_indexes[1]
    NPAD = -(-N // 256) * 256

    C4 = jnp.stack([Wv[:DN, 0], Wv[DN:, 0], We[:DN, 0], We[DN:2 * DN, 0]],
                   axis=1)
    P = W1 @ C4
    q = b1 @ C4
    q = q.at[0].add(bv[0])
    q = q.at[2].add(b2 @ We[2 * DN:, 0] + be[0])
    P_pad = jnp.zeros((DN, 128), jnp.float32).at[:, :4].set(P)
    q_pad = jnp.zeros((1, 128), jnp.float32).at[0, :4].set(q)
    w2e = W2 @ We[2 * DN:, 0]
    A = W1 @ W3[:DN]
    B = W2 @ W3[DN:DN + DE]
    W3c = W3[DN + DE:]
    c0 = (b1 @ W3[:DN] + b2 @ W3[DN:DN + DE] + b3)[None, :]

    gsel = jnp.arange(128) // DE
    Mw = jnp.where(gsel[:, None] == jnp.arange(8)[None, :],
                   jnp.tile(w2e, 8)[:, None], 0.0)
    ef2 = edge_features.reshape(E // 8, 128)
    S_full, eE8 = pl.pallas_call(
        _k1_body,
        out_shape=[
            jax.ShapeDtypeStruct((N, 128), jnp.float32),
            jax.ShapeDtypeStruct((E // 8, 8), jnp.float32),
        ],
    )(node_features, P_pad, q_pad, ef2, Mw)
    s0, s1, s2, s3 = (S_full[:, 0], S_full[:, 1], S_full[:, 2], S_full[:, 3])
    eE = eE8.reshape(E)

    zstripe = jnp.zeros((NPAD // _NS, 128), jnp.float32)
    evw, acc2 = _make_sc_scalar_kernel(N, E, DN, DE, NPAD)(
        edge_features, u, v, s0, s1, s2, s3, eE, zstripe)
    acc1 = _make_sc_rows_kernel(N, E, DN, NPAD)(
        node_features, u, v, evw, zstripe)
    acc2r = acc2.reshape(_NC, NPAD, 32)

    out = pl.pallas_call(
        _k2_body,
        out_shape=jax.ShapeDtypeStruct((N, DO), jnp.float32),
    )(acc1[0], acc1[1], acc2r[0], acc2r[1], node_features, A, B, W3c, c0)
    return out


# speedup vs baseline: 21.3559x; 1.1623x over previous
import functools

import jax
import jax.numpy as jnp
from jax import lax
from jax.experimental import pallas as pl
from jax.experimental.pallas import tpu as pltpu
from jax.experimental.pallas import tpu_sc as plsc

_NC = 2
_NS = 16
_LANES = 16


def _k1_body(nf_ref, p_ref, q_ref, ef_ref, mw_ref, s_ref, e_ref):
    s_ref[...] = (
        jnp.dot(nf_ref[...], p_ref[...], preferred_element_type=jnp.float32)
        + q_ref[...]
    )
    e_ref[...] = jnp.dot(ef_ref[...], mw_ref[...],
                         preferred_element_type=jnp.float32)


def _k2_body(a10_ref, a11_ref, a20_ref, a21_ref, nf_ref,
             A_ref, B_ref, Wc_ref, c0_ref, o_ref):
    n = nf_ref.shape[0]
    accV = a10_ref[:n, :] + a11_ref[:n, :]
    s2 = a20_ref[:n, :] + a21_ref[:n, :]
    accE = s2[:, :16]
    dV = jnp.maximum(s2[:, 16:17], 1e-30)
    dE = jnp.maximum(s2[:, 17:18], 1e-30)
    deg = s2[:, 18:19]
    emb = (
        jnp.dot(accV / dV, A_ref[...], preferred_element_type=jnp.float32)
        + jnp.dot(accE / dE, B_ref[...], preferred_element_type=jnp.float32)
        + deg * (jnp.dot(nf_ref[...], Wc_ref[...],
                         preferred_element_type=jnp.float32) + c0_ref[...])
    )
    m = jnp.max(emb, axis=1, keepdims=True)
    ex = jnp.exp(emb - m)
    o_ref[...] = ex / jnp.sum(ex, axis=1, keepdims=True)


def _make_sc_scalar_kernel(N, E, DN, DE, NPAD):
    NT = _NC * _NS
    EP = E // NT
    C = 16
    NCHUNK = EP // C
    SR2 = NPAD // _NS

    mesh = plsc.VectorSubcoreMesh(core_axis_name="c", subcore_axis_name="s")

    NB = 4
    NH = NCHUNK // NB

    def body(ef, u, v, s0, s1, s2, s3, ee, zst, z2, evw, acc2,
             u_v, v_v, ee_v, s0_v, s1_v, s2_v, s3_v,
             ev_slab, pay2, ef_v, es0, es1, es2, es3,
             ps0, ps1, ps2, ps3, acc2_sh):
        cid = lax.axis_index("c")
        sid = lax.axis_index("s")
        base = (cid * _NS + sid) * EP
        efsems = (es0, es1, es2, es3)
        psems = (ps0, ps1, ps2, ps3)

        pltpu.sync_copy(z2, acc2_sh.at[pl.ds(sid * SR2, SR2)])

        pltpu.sync_copy(u.at[pl.ds(base, EP)], u_v)
        pltpu.sync_copy(v.at[pl.ds(base, EP)], v_v)
        pltpu.sync_copy(ee.at[pl.ds(base, EP)], ee_v)
        pltpu.sync_copy(s0, s0_v)
        pltpu.sync_copy(s1, s1_v)
        pltpu.sync_copy(s2, s2_v)
        pltpu.sync_copy(s3, s3_v)

        zv = jnp.zeros((_LANES,), jnp.float32)
        for b in range(4):
            for j in range(C):
                for r in range(2):
                    pay2[b, j, pl.ds(r * _LANES, _LANES)] = zv

        plsc.subcore_barrier()

        row_iota = lax.iota(jnp.int32, _LANES)
        ones16 = jnp.ones((_LANES,), jnp.float32)

        for b in range(4):
            pltpu.async_copy(ef.at[pl.ds(base + b * C, C)],
                             ef_v.at[b], efsems[b])

        def pair(kk, carry):
            for b in range(4):
                off = (kk * 4 + b) * C
                u16 = u_v[pl.ds(off, _LANES)]
                v16 = v_v[pl.ds(off, _LANES)]
                sVu = plsc.load_gather(s0_v, [u16])
                sVv = plsc.load_gather(s1_v, [v16])
                sEu = plsc.load_gather(s2_v, [u16])
                sEv = plsc.load_gather(s3_v, [v16])
                ee16 = ee_v[pl.ds(off, _LANES)]
                aV = sVu + sVv
                aV = jnp.where(aV >= 0.0, aV, aV * 0.01)
                aE = sEu + sEv + ee16
                aE = jnp.where(aE >= 0.0, aE, aE * 0.01)
                eV = jnp.exp(aV)
                eEx = jnp.exp(aE)
                ev_slab[pl.ds(off, _LANES)] = eV

                pltpu.make_async_copy(ef.at[pl.ds(0, C)],
                                      ef_v.at[b], efsems[b]).wait()

                @pl.when(kk >= 1)
                def _():
                    pltpu.make_async_copy(acc2.at[0, pl.ds(0, C)],
                                          pay2.at[b], psems[b]).wait()

                for f in range(DE):
                    colf = jnp.full((_LANES,), f, jnp.int32)
                    efc = plsc.load_gather(ef_v.at[b], [row_iota, colf])
                    plsc.store_scatter(pay2.at[b], [row_iota, colf],
                                       efc * eEx)
                plsc.store_scatter(pay2.at[b],
                                   [row_iota, jnp.full((_LANES,), DE, jnp.int32)], eV)
                plsc.store_scatter(pay2.at[b],
                                   [row_iota, jnp.full((_LANES,), DE + 1, jnp.int32)], eEx)
                plsc.store_scatter(pay2.at[b],
                                   [row_iota, jnp.full((_LANES,), DE + 2, jnp.int32)], ones16)

                pltpu.async_copy(pay2.at[b], acc2_sh.at[u16], psems[b],
                                 add=True)

                @pl.when(kk < NH - 1)
                def _():
                    pltpu.async_copy(ef.at[pl.ds(base + off + 4 * C, C)],
                                     ef_v.at[b], efsems[b])
            return carry

        lax.fori_loop(0, NH, pair, 0)

        for b in range(4):
            pltpu.make_async_copy(acc2.at[0, pl.ds(0, C)],
                                  pay2.at[b], psems[b]).wait()

        for k in range(NH * NB, NCHUNK):
            off = k * C
            u16 = u_v[pl.ds(off, _LANES)]
            v16 = v_v[pl.ds(off, _LANES)]
            sVu = plsc.load_gather(s0_v, [u16])
            sVv = plsc.load_gather(s1_v, [v16])
            sEu = plsc.load_gather(s2_v, [u16])
            sEv = plsc.load_gather(s3_v, [v16])
            ee16 = ee_v[pl.ds(off, _LANES)]
            aV = sVu + sVv
            aV = jnp.where(aV >= 0.0, aV, aV * 0.01)
            aE = sEu + sEv + ee16
            aE = jnp.where(aE >= 0.0, aE, aE * 0.01)
            eV = jnp.exp(aV)
            eEx = jnp.exp(aE)
            ev_slab[pl.ds(off, _LANES)] = eV
            b = k % NB
            pltpu.sync_copy(ef.at[pl.ds(base + off, C)], ef_v.at[b])
            for f in range(DE):
                colf = jnp.full((_LANES,), f, jnp.int32)
                efc = plsc.load_gather(ef_v.at[b], [row_iota, colf])
                plsc.store_scatter(pay2.at[b], [row_iota, colf],
                                   efc * eEx)
            plsc.store_scatter(pay2.at[b],
                               [row_iota, jnp.full((_LANES,), DE, jnp.int32)], eV)
            plsc.store_scatter(pay2.at[b],
                               [row_iota, jnp.full((_LANES,), DE + 1, jnp.int32)], eEx)
            plsc.store_scatter(pay2.at[b],
                               [row_iota, jnp.full((_LANES,), DE + 2, jnp.int32)], ones16)
            pltpu.sync_copy(pay2.at[b], acc2_sh.at[u16], add=True)

        pltpu.sync_copy(ev_slab, evw.at[pl.ds(base, EP)])
        plsc.subcore_barrier()
        pltpu.sync_copy(acc2_sh.at[pl.ds(sid * SR2, SR2)],
                        acc2.at[cid, pl.ds(sid * SR2, SR2)])

    return functools.partial(
        pl.kernel,
        out_type=(
            jax.ShapeDtypeStruct((E,), jnp.float32),
            jax.ShapeDtypeStruct((_NC, NPAD, 32), jnp.float32),
        ),
        mesh=mesh,
        compiler_params=pltpu.CompilerParams(needs_layout_passes=False,
                                             use_tc_tiling_on_sc=False),
        scratch_types=[
            pltpu.VMEM((EP,), jnp.int32),
            pltpu.VMEM((EP,), jnp.int32),
            pltpu.VMEM((EP,), jnp.float32),
            pltpu.VMEM((N,), jnp.float32),
            pltpu.VMEM((N,), jnp.float32),
            pltpu.VMEM((N,), jnp.float32),
            pltpu.VMEM((N,), jnp.float32),
            pltpu.VMEM((EP,), jnp.float32),
            pltpu.VMEM((4, C, 32), jnp.float32),
            pltpu.VMEM((4, C, DE), jnp.float32),
            pltpu.SemaphoreType.DMA,
            pltpu.SemaphoreType.DMA,
            pltpu.SemaphoreType.DMA,
            pltpu.SemaphoreType.DMA,
            pltpu.SemaphoreType.DMA,
            pltpu.SemaphoreType.DMA,
            pltpu.SemaphoreType.DMA,
            pltpu.SemaphoreType.DMA,
            pltpu.VMEM_SHARED((NPAD, 32), jnp.float32),
        ],
    )(body)


def _make_sc_rows_kernel(N, E, DN, NPAD):
    NT = _NC * _NS
    EP = E // NT
    C = 16
    NCHUNK = EP // C
    SR1 = NPAD // _NS

    mesh = plsc.VectorSubcoreMesh(core_axis_name="c", subcore_axis_name="s")

    NB = 4
    NH = NCHUNK // NB

    def body(nf, u, v, evw, zst, acc1,
             u_v, v_v, ev_slab, rows, pay, ev_v,
             gs0, gs1, gs2, gs3, ss0, ss1, ss2, ss3, acc1_sh):
        cid = lax.axis_index("c")
        sid = lax.axis_index("s")
        base = (cid * _NS + sid) * EP
        gsems = (gs0, gs1, gs2, gs3)
        ssems = (ss0, ss1, ss2, ss3)

        pltpu.sync_copy(zst, acc1_sh.at[pl.ds(sid * SR1, SR1)])
        pltpu.sync_copy(u.at[pl.ds(base, EP)], u_v)
        pltpu.sync_copy(v.at[pl.ds(base, EP)], v_v)
        pltpu.sync_copy(evw.at[pl.ds(base, EP)], ev_slab)
        plsc.subcore_barrier()

        for b in range(4):
            pltpu.async_copy(nf.at[v_v[pl.ds(b * C, _LANES)]],
                             rows.at[b], gsems[b])

        def pair(kk, carry):
            for b in range(4):
                off = (kk * 4 + b) * C
                u16 = u_v[pl.ds(off, _LANES)]
                eV = ev_slab[pl.ds(off, _LANES)]
                pltpu.make_async_copy(nf.at[pl.ds(0, C)],
                                      rows.at[b], gsems[b]).wait()

                @pl.when(kk >= 1)
                def _():
                    pltpu.make_async_copy(nf.at[pl.ds(0, C)],
                                          pay.at[b], ssems[b]).wait()

                ev_v[...] = eV
                for j in range(C):
                    sv = plsc.load_gather(
                        ev_v, [jnp.full((_LANES,), j, jnp.int32)])
                    for r in range(DN // _LANES):
                        pay[b, j, pl.ds(r * _LANES, _LANES)] = (
                            rows[b, j, pl.ds(r * _LANES, _LANES)] * sv)
                pltpu.async_copy(pay.at[b], acc1_sh.at[u16], ssems[b],
                                 add=True)

                @pl.when(kk < NH - 1)
                def _():
                    v16n = v_v[pl.ds(off + 4 * C, _LANES)]
                    pltpu.async_copy(nf.at[v16n], rows.at[b], gsems[b])
            return carry

        lax.fori_loop(0, NH, pair, 0)

        for b in range(4):
            pltpu.make_async_copy(nf.at[pl.ds(0, C)],
                                  pay.at[b], ssems[b]).wait()

        for k in range(NH * NB, NCHUNK):
            off = k * C
            u16 = u_v[pl.ds(off, _LANES)]
            v16 = v_v[pl.ds(off, _LANES)]
            eV = ev_slab[pl.ds(off, _LANES)]
            b = k % NB
            pltpu.sync_copy(nf.at[v16], rows.at[b])
            ev_v[...] = eV
            for j in range(C):
                sv = plsc.load_gather(
                    ev_v, [jnp.full((_LANES,), j, jnp.int32)])
                for r in range(DN // _LANES):
                    pay[b, j, pl.ds(r * _LANES, _LANES)] = (
                        rows[b, j, pl.ds(r * _LANES, _LANES)] * sv)
            pltpu.sync_copy(pay.at[b], acc1_sh.at[u16], add=True)

        plsc.subcore_barrier()
        pltpu.sync_copy(acc1_sh.at[pl.ds(sid * SR1, SR1)],
                        acc1.at[cid, pl.ds(sid * SR1, SR1)])

    return functools.partial(
        pl.kernel,
        out_type=jax.ShapeDtypeStruct((_NC, NPAD, 128), jnp.float32),
        mesh=mesh,
        compiler_params=pltpu.CompilerParams(needs_layout_passes=False),
        scratch_types=[
            pltpu.VMEM((EP,), jnp.int32),
            pltpu.VMEM((EP,), jnp.int32),
            pltpu.VMEM((EP,), jnp.float32),
            pltpu.VMEM((4, C, DN), jnp.float32),
            pltpu.VMEM((4, C, DN), jnp.float32),
            pltpu.VMEM((_LANES,), jnp.float32),
            pltpu.SemaphoreType.DMA,
            pltpu.SemaphoreType.DMA,
            pltpu.SemaphoreType.DMA,
            pltpu.SemaphoreType.DMA,
            pltpu.SemaphoreType.DMA,
            pltpu.SemaphoreType.DMA,
            pltpu.SemaphoreType.DMA,
            pltpu.SemaphoreType.DMA,
            pltpu.VMEM_SHARED((NPAD, 128), jnp.float32),
        ],
    )(body)


def kernel(node_features, edge_features, edge_indexes,
           W1, b1, W2, b2, W3, b3, Wv, bv, We, be):
    N, DN = node_features.shape
    E, DE = edge_features.shape
    DO = W3.shape[1]
    u = edge_indexes[0]
    v = edge_indexes[1]
    NPAD = -(-N // 256) * 256

    C4 = jnp.stack([Wv[:DN, 0], Wv[DN:, 0], We[:DN, 0], We[DN:2 * DN, 0]],
                   axis=1)
    P = W1 @ C4
    q = b1 @ C4
    q = q.at[0].add(bv[0])
    q = q.at[2].add(b2 @ We[2 * DN:, 0] + be[0])
    P_pad = jnp.zeros((DN, 128), jnp.float32).at[:, :4].set(P)
    q_pad = jnp.zeros((1, 128), jnp.float32).at[0, :4].set(q)
    w2e = W2 @ We[2 * DN:, 0]
    A = W1 @ W3[:DN]
    B = W2 @ W3[DN:DN + DE]
    W3c = W3[DN + DE:]
    c0 = (b1 @ W3[:DN] + b2 @ W3[DN:DN + DE] + b3)[None, :]

    gsel = jnp.arange(128) // DE
    Mw = jnp.where(gsel[:, None] == jnp.arange(8)[None, :],
                   jnp.tile(w2e, 8)[:, None], 0.0)
    ef2 = edge_features.reshape(E // 8, 128)
    S_full, eE8 = pl.pallas_call(
        _k1_body,
        out_shape=[
            jax.ShapeDtypeStruct((N, 128), jnp.float32),
            jax.ShapeDtypeStruct((E // 8, 8), jnp.float32),
        ],
    )(node_features, P_pad, q_pad, ef2, Mw)
    s0, s1, s2, s3 = (S_full[:, 0], S_full[:, 1], S_full[:, 2], S_full[:, 3])
    eE = eE8.reshape(E)

    zstripe = jnp.zeros((NPAD // _NS, 128), jnp.float32)
    z2stripe = jnp.zeros((NPAD // _NS, 32), jnp.float32)
    evw, acc2r = _make_sc_scalar_kernel(N, E, DN, DE, NPAD)(
        edge_features, u, v, s0, s1, s2, s3, eE, zstripe, z2stripe)
    acc1 = _make_sc_rows_kernel(N, E, DN, NPAD)(
        node_features, u, v, evw, zstripe)

    out = pl.pallas_call(
        _k2_body,
        out_shape=jax.ShapeDtypeStruct((N, DO), jnp.float32),
    )(acc1[0], acc1[1], acc2r[0], acc2r[1], node_features, A, B, W3c, c0)
    return out
